# baseline (speedup 1.0000x reference)
import functools

import jax
import jax.numpy as jnp
from jax import lax
from jax.experimental import pallas as pl
from jax.experimental.pallas import tpu as pltpu
from jax.experimental.pallas import tpu_sc as plsc

TOPK = 8
_HIGHEST = jax.lax.Precision.HIGHEST


def _xk_body(x_ref, xp_ref, tmk_ref, xk_ref):
    x = x_ref[...]
    xk_ref[...] = (x + (xp_ref[...] - x) * tmk_ref[...]).astype(jnp.bfloat16)


def _qsim_body(xk_ref, wq_ref, kt_ref, sim_ref):
    q = jax.lax.dot_general(
        xk_ref[...], wq_ref[0], (((1,), (1,)), ((), ())),
        preferred_element_type=jnp.float32)
    sim_ref[...] = jax.lax.dot_general(
        q.astype(jnp.bfloat16), kt_ref[0], (((1,), (1,)), ((), ())),
        preferred_element_type=jnp.float32)[None]


def _select_body(sim_ref, idx_ref, w_ref, *, heads, num_keys):
    s = sim_ref[...]
    tb = s.shape[1]
    iota = jax.lax.broadcasted_iota(jnp.int32, s.shape, 2)
    vs, is_ = [], []
    v = s
    for _ in range(TOPK):
        m = jnp.max(v, axis=-1, keepdims=True)
        am = jnp.min(jnp.where(v >= m, iota, jnp.int32(1 << 30)),
                     axis=-1, keepdims=True)
        vs.append(m)
        is_.append(am)
        v = jnp.where(iota == am, -jnp.inf, v)
    sv = jnp.concatenate(vs, -1)
    si = jnp.concatenate(is_, -1)
    sx, ix = sv[:heads], si[:heads]
    sy, iy = sv[heads:], si[heads:]
    alls = (sx[..., :, None] + sy[..., None, :]).reshape(heads, tb, TOPK * TOPK)
    iota2 = jax.lax.broadcasted_iota(
        jnp.int32, (heads, tb, TOPK * TOPK), 2)
    iota8 = jax.lax.broadcasted_iota(jnp.int32, (heads, tb, TOPK), 2)
    vs, ps = [], []
    v = alls
    for _ in range(TOPK):
        m = jnp.max(v, axis=-1, keepdims=True)
        am = jnp.min(jnp.where(v >= m, iota2, jnp.int32(1 << 30)),
                     axis=-1, keepdims=True)
        hi = am >> 3
        lo = am & 7
        selx = jnp.sum(jnp.where(iota8 == hi, ix, 0), axis=-1, keepdims=True)
        sely = jnp.sum(jnp.where(iota8 == lo, iy, 0), axis=-1, keepdims=True)
        vs.append(m)
        ps.append(selx * num_keys + sely)
        v = jnp.where(iota2 == am, -jnp.inf, v)
    scores = jnp.concatenate(vs, -1)
    sel = jnp.concatenate(ps, -1)
    mx = jnp.max(scores, -1, keepdims=True)
    e = jnp.exp(scores - mx)
    w = e / jnp.sum(e, -1, keepdims=True)
    idx_ref[...] = sel
    w_ref[...] = w


def _wsum_body(idx_ref, w_ref, out_ref, *, num_keys, g):
    heads = idx_ref.shape[0]
    rows = heads * g * TOPK
    cols = g * num_keys
    idx = idx_ref[...]
    w = w_ref[...]
    a = idx // num_keys
    b = idx - a * num_keys
    t_iota = jax.lax.broadcasted_iota(jnp.int32, idx.shape, 1)
    tgt = (t_iota * num_keys + a)[..., None]
    ci = jax.lax.broadcasted_iota(jnp.int32, (heads, g, TOPK, cols), 3)
    u2 = jnp.where(ci == tgt, w[..., None], 0.0).reshape(rows, cols)
    ji = jax.lax.broadcasted_iota(jnp.int32, (heads, g, TOPK, num_keys), 3)
    v2 = jnp.where(ji == b[..., None], 1.0, 0.0).reshape(rows, num_keys)
    m = jax.lax.dot_general(
        u2, v2, (((0,), (0,)), ((), ())),
        preferred_element_type=jnp.float32)
    out_ref[...] = m


def _dedup_body(idx_ref, w_ref, out_ref):
    idx = idx_ref[...]
    w = w_ref[...]
    eq = idx[:, :, None] == idx[:, None, :]
    out_ref[...] = jnp.sum(jnp.where(eq, w[:, None, :], 0.0), axis=2)


def _make_sc_scatter(tokens, slots, num_experts):
    info = plsc.get_sparse_core_info()
    nw = info.num_cores * info.num_subcores
    lanes = info.num_lanes
    tpw = tokens // nw
    width = num_experts + lanes
    chunks = slots // lanes
    ring = 8
    mesh = plsc.VectorSubcoreMesh(core_axis_name="c", subcore_axis_name="s")

    @functools.partial(
        pl.kernel, mesh=mesh,
        compiler_params=pltpu.CompilerParams(needs_layout_passes=False),
        out_type=jax.ShapeDtypeStruct((tokens * num_experts,), jnp.float32),
        scratch_types=[
            pltpu.VMEM((tpw, slots), jnp.int32),
            pltpu.VMEM((tpw, slots), jnp.float32),
            pltpu.VMEM((ring * width,), jnp.float32),
            pltpu.SemaphoreType.DMA((ring,)),
        ],
    )
    def scat(idx_hbm, w_hbm, out_hbm, idx_v, w_v, row_v, sem):
        wid = lax.axis_index("s") * info.num_cores + lax.axis_index("c")
        base = wid * tpw
        pltpu.sync_copy(idx_hbm.at[pl.ds(base, tpw)], idx_v)
        pltpu.sync_copy(w_hbm.at[pl.ds(base, tpw)], w_v)
        zeros = jnp.zeros((lanes,), jnp.float32)

        def zero_body(i, carry):
            row_v[pl.ds(i * lanes, lanes)] = zeros
            return carry

        lax.fori_loop(0, ring * width // lanes, zero_body, 0)

        def fire(t, slot):
            for c in range(chunks):
                vi = idx_v[t, pl.ds(c * lanes, lanes)] + slot * width
                vw = w_v[t, pl.ds(c * lanes, lanes)]
                plsc.store_scatter(row_v, [vi], vw)
            pltpu.async_copy(
                row_v.at[pl.ds(slot * width, num_experts)],
                out_hbm.at[pl.ds((base + t) * num_experts, num_experts)],
                sem.at[slot])

        def tok_body(t, carry):
            slot = lax.rem(t, ring)
            pltpu.make_async_copy(
                row_v.at[pl.ds(slot * width, num_experts)],
                out_hbm.at[pl.ds(base * num_experts, num_experts)],
                sem.at[slot]).wait()
            for c in range(chunks):
                vi = idx_v[t - ring, pl.ds(c * lanes, lanes)] + slot * width
                plsc.store_scatter(row_v, [vi], zeros)
            fire(t, slot)
            return carry

        def prime_body(t, carry):
            fire(t, lax.rem(t, ring))
            return carry

        lax.fori_loop(0, ring, prime_body, 0)
        lax.fori_loop(ring, tpw, tok_body, 0)

        def drain_body(s, carry):
            pltpu.make_async_copy(
                row_v.at[pl.ds(s * width, num_experts)],
                out_hbm.at[pl.ds(base * num_experts, num_experts)],
                sem.at[s]).wait()
            return carry

        lax.fori_loop(0, ring, drain_body, 0)

    return scat


def _gelu_h_body(xk_ref, wd_ref, g_ref):
    h = jax.lax.dot_general(
        xk_ref[...], wd_ref[...], (((1,), (1,)), ((), ())),
        preferred_element_type=jnp.float32)
    g_ref[...] = 0.5 * h * (1.0 + jax.lax.erf(h * (2.0 ** -0.5)))


def _mlp_body(g_ref, wu_ref, ws_ref, out_ref, acc_ref, *, ne_blocks):
    e = pl.program_id(0)
    t = pl.program_id(1)
    ws3 = ws_ref[...]
    gate = g_ref[...] * ws3.reshape(ws3.shape[0], ws3.shape[1] * ws3.shape[2])
    contrib = jax.lax.dot_general(
        gate.astype(jnp.bfloat16), wu_ref[...], (((1,), (0,)), ((), ())),
        preferred_element_type=jnp.float32)
    tb = g_ref.shape[0]
    sl = pl.ds(t * tb, tb)

    @pl.when(e == 0)
    def _():
        acc_ref[sl, :] = contrib

    @pl.when(e > 0)
    def _():
        acc_ref[sl, :] = acc_ref[sl, :] + contrib

    @pl.when(e == ne_blocks - 1)
    def _():
        out_ref[...] = acc_ref[sl, :]


def kernel(x, shift_state, time_maa_k, time_maa_r, W_q, keys_p, w_down, w_up):
    del time_maa_r
    bsz, t_len, n_embd = x.shape
    heads, num_keys, _, dim_key = keys_p.shape
    num_experts = num_keys * num_keys

    new_shift_state = x[:, -1]
    xprev = jnp.concatenate([shift_state[:, None, :], x[:, :-1]], axis=1)
    x2 = x.reshape(bsz * t_len, n_embd)
    xp2 = xprev.reshape(bsz * t_len, n_embd)
    tmk = time_maa_k.reshape(1, n_embd)
    tokens = bsz * t_len

    wqr = W_q.reshape(2 * heads, dim_key, n_embd).astype(jnp.bfloat16)
    kt = keys_p.transpose(2, 0, 1, 3).reshape(
        2 * heads, num_keys, dim_key).astype(jnp.bfloat16)
    wd_b = w_down.astype(jnp.bfloat16)
    wu_b = w_up.astype(jnp.bfloat16)

    tb = 256 if tokens % 256 == 0 else tokens
    nt = tokens // tb
    xk = pl.pallas_call(
        _xk_body,
        grid=(nt,),
        in_specs=[
            pl.BlockSpec((tb, n_embd), lambda i: (i, 0)),
            pl.BlockSpec((tb, n_embd), lambda i: (i, 0)),
            pl.BlockSpec((1, n_embd), lambda i: (0, 0)),
        ],
        out_specs=pl.BlockSpec((tb, n_embd), lambda i: (i, 0)),
        out_shape=jax.ShapeDtypeStruct((tokens, n_embd), jnp.bfloat16),
    )(x2, xp2, tmk)

    sim = pl.pallas_call(
        _qsim_body,
        grid=(2 * heads, nt),
        in_specs=[
            pl.BlockSpec((tb, n_embd), lambda p, t: (t, 0)),
            pl.BlockSpec((1, dim_key, n_embd), lambda p, t: (p, 0, 0)),
            pl.BlockSpec((1, num_keys, dim_key), lambda p, t: (p, 0, 0)),
        ],
        out_specs=pl.BlockSpec((1, tb, num_keys), lambda p, t: (p, t, 0)),
        out_shape=jax.ShapeDtypeStruct((2 * heads, tokens, num_keys),
                                       jnp.float32),
        compiler_params=pltpu.CompilerParams(
            dimension_semantics=("arbitrary", "arbitrary")),
    )(xk, wqr, kt)

    idx, wts = pl.pallas_call(
        functools.partial(_select_body, heads=heads, num_keys=num_keys),
        grid=(nt,),
        in_specs=[pl.BlockSpec((2 * heads, tb, num_keys), lambda i: (0, i, 0))],
        out_specs=[
            pl.BlockSpec((heads, tb, TOPK), lambda i: (0, i, 0)),
            pl.BlockSpec((heads, tb, TOPK), lambda i: (0, i, 0)),
        ],
        out_shape=[
            jax.ShapeDtypeStruct((heads, tokens, TOPK), jnp.int32),
            jax.ShapeDtypeStruct((heads, tokens, TOPK), jnp.float32),
        ],
    )(sim)

    slots = heads * TOPK
    idx2 = idx.transpose(1, 0, 2).reshape(tokens, slots)
    wts2 = wts.transpose(1, 0, 2).reshape(tokens, slots)
    wts2 = pl.pallas_call(
        _dedup_body,
        grid=(nt,),
        in_specs=[
            pl.BlockSpec((tb, slots), lambda i: (i, 0)),
            pl.BlockSpec((tb, slots), lambda i: (i, 0)),
        ],
        out_specs=pl.BlockSpec((tb, slots), lambda i: (i, 0)),
        out_shape=jax.ShapeDtypeStruct((tokens, slots), jnp.float32),
    )(idx2, wts2)
    scat = _make_sc_scatter(tokens, slots, num_experts)
    wsum3 = scat(idx2, wts2).reshape(tokens, num_keys, num_keys)

    ne_blocks = 8
    eb = num_experts // ne_blocks
    g0 = pl.pallas_call(
        _gelu_h_body,
        grid=(ne_blocks, nt),
        in_specs=[
            pl.BlockSpec((tb, n_embd), lambda e, t: (t, 0)),
            pl.BlockSpec((eb, n_embd), lambda e, t: (e, 0)),
        ],
        out_specs=pl.BlockSpec((tb, eb), lambda e, t: (t, e)),
        out_shape=jax.ShapeDtypeStruct((tokens, num_experts), jnp.float32),
        compiler_params=pltpu.CompilerParams(
            dimension_semantics=("arbitrary", "arbitrary")),
    )(xk, wd_b)

    out = pl.pallas_call(
        functools.partial(_mlp_body, ne_blocks=ne_blocks),
        grid=(ne_blocks, nt),
        in_specs=[
            pl.BlockSpec((tb, eb), lambda e, t: (t, e)),
            pl.BlockSpec((eb, n_embd), lambda e, t: (e, 0)),
            pl.BlockSpec((tb, eb // num_keys, num_keys),
                         lambda e, t: (t, e, 0)),
        ],
        out_specs=pl.BlockSpec(
            (tb, n_embd),
            lambda e, t: (jnp.where(e == ne_blocks - 1, t, 0), 0)),
        out_shape=jax.ShapeDtypeStruct((tokens, n_embd), jnp.float32),
        scratch_shapes=[pltpu.VMEM((tokens, n_embd), jnp.float32)],
        compiler_params=pltpu.CompilerParams(
            dimension_semantics=("arbitrary", "arbitrary")),
    )(g0, wu_b, wsum3)

    return out.reshape(bsz, t_len, n_embd), new_shift_state


# speedup vs baseline: 1.0744x; 1.0744x over previous
import functools

import jax
import jax.numpy as jnp
from jax import lax
from jax.experimental import pallas as pl
from jax.experimental.pallas import tpu as pltpu
from jax.experimental.pallas import tpu_sc as plsc

TOPK = 8
_HIGHEST = jax.lax.Precision.HIGHEST


def _xk_body(x_ref, xp_ref, tmk_ref, xk_ref):
    x = x_ref[...]
    xk_ref[...] = (x + (xp_ref[...] - x) * tmk_ref[...]).astype(jnp.bfloat16)


def _qsim_body(xk_ref, wq_ref, kt_ref, sim_ref):
    q = jax.lax.dot_general(
        xk_ref[...], wq_ref[0], (((1,), (1,)), ((), ())),
        preferred_element_type=jnp.float32)
    sim_ref[...] = jax.lax.dot_general(
        q.astype(jnp.bfloat16), kt_ref[0], (((1,), (1,)), ((), ())),
        preferred_element_type=jnp.float32)[None]


def _select_body(sim_ref, idx_ref, w_ref, *, heads, num_keys):
    s = sim_ref[...]
    tb = s.shape[1]
    iota = jax.lax.broadcasted_iota(jnp.int32, s.shape, 2)
    vs, is_ = [], []
    v = s
    for _ in range(TOPK):
        m = jnp.max(v, axis=-1, keepdims=True)
        am = jnp.min(jnp.where(v >= m, iota, jnp.int32(1 << 30)),
                     axis=-1, keepdims=True)
        vs.append(m)
        is_.append(am)
        v = jnp.where(iota == am, -jnp.inf, v)
    sv = jnp.concatenate(vs, -1)
    si = jnp.concatenate(is_, -1)
    sx, ix = sv[:heads], si[:heads]
    sy, iy = sv[heads:], si[heads:]
    alls = (sx[..., :, None] + sy[..., None, :]).reshape(heads, tb, TOPK * TOPK)
    iota2 = jax.lax.broadcasted_iota(
        jnp.int32, (heads, tb, TOPK * TOPK), 2)
    iota8 = jax.lax.broadcasted_iota(jnp.int32, (heads, tb, TOPK), 2)
    vs, ps = [], []
    v = alls
    for _ in range(TOPK):
        m = jnp.max(v, axis=-1, keepdims=True)
        am = jnp.min(jnp.where(v >= m, iota2, jnp.int32(1 << 30)),
                     axis=-1, keepdims=True)
        hi = am >> 3
        lo = am & 7
        selx = jnp.sum(jnp.where(iota8 == hi, ix, 0), axis=-1, keepdims=True)
        sely = jnp.sum(jnp.where(iota8 == lo, iy, 0), axis=-1, keepdims=True)
        vs.append(m)
        ps.append(selx * num_keys + sely)
        v = jnp.where(iota2 == am, -jnp.inf, v)
    scores = jnp.concatenate(vs, -1)
    sel = jnp.concatenate(ps, -1)
    mx = jnp.max(scores, -1, keepdims=True)
    e = jnp.exp(scores - mx)
    w = e / jnp.sum(e, -1, keepdims=True)
    idx_ref[...] = sel
    w_ref[...] = w


def _wsum_body(idx_ref, w_ref, out_ref, *, num_keys, g):
    heads = idx_ref.shape[0]
    rows = heads * g * TOPK
    cols = g * num_keys
    idx = idx_ref[...]
    w = w_ref[...]
    a = idx // num_keys
    b = idx - a * num_keys
    t_iota = jax.lax.broadcasted_iota(jnp.int32, idx.shape, 1)
    tgt = (t_iota * num_keys + a)[..., None]
    ci = jax.lax.broadcasted_iota(jnp.int32, (heads, g, TOPK, cols), 3)
    u2 = jnp.where(ci == tgt, w[..., None], 0.0).reshape(rows, cols)
    ji = jax.lax.broadcasted_iota(jnp.int32, (heads, g, TOPK, num_keys), 3)
    v2 = jnp.where(ji == b[..., None], 1.0, 0.0).reshape(rows, num_keys)
    m = jax.lax.dot_general(
        u2, v2, (((0,), (0,)), ((), ())),
        preferred_element_type=jnp.float32)
    out_ref[...] = m


def _dedup_body(idx_ref, w_ref, out_ref):
    idx = idx_ref[...]
    w = w_ref[...]
    eq = idx[:, :, None] == idx[:, None, :]
    out_ref[...] = jnp.sum(jnp.where(eq, w[:, None, :], 0.0), axis=2)


def _make_sc_scatter(tokens, slots, num_experts):
    info = plsc.get_sparse_core_info()
    nw = info.num_cores * info.num_subcores
    lanes = info.num_lanes
    tpw = tokens // nw
    width = num_experts + lanes
    chunks = slots // lanes
    ring = 8
    mesh = plsc.VectorSubcoreMesh(core_axis_name="c", subcore_axis_name="s")

    @functools.partial(
        pl.kernel, mesh=mesh,
        compiler_params=pltpu.CompilerParams(needs_layout_passes=False),
        out_type=jax.ShapeDtypeStruct((tokens * num_experts,), jnp.float32),
        scratch_types=[
            pltpu.VMEM((tpw, slots), jnp.int32),
            pltpu.VMEM((tpw, slots), jnp.float32),
            pltpu.VMEM((ring * width,), jnp.float32),
            pltpu.SemaphoreType.DMA((ring,)),
        ],
    )
    def scat(idx_hbm, w_hbm, out_hbm, idx_v, w_v, row_v, sem):
        wid = lax.axis_index("s") * info.num_cores + lax.axis_index("c")
        base = wid * tpw
        pltpu.sync_copy(idx_hbm.at[pl.ds(base, tpw)], idx_v)
        pltpu.sync_copy(w_hbm.at[pl.ds(base, tpw)], w_v)
        zeros = jnp.zeros((lanes,), jnp.float32)

        def zero_body(i, carry):
            row_v[pl.ds(i * lanes, lanes)] = zeros
            return carry

        lax.fori_loop(0, ring * width // lanes, zero_body, 0)

        def fire(t, slot):
            for c in range(chunks):
                vi = idx_v[t, pl.ds(c * lanes, lanes)] + slot * width
                vw = w_v[t, pl.ds(c * lanes, lanes)]
                plsc.store_scatter(row_v, [vi], vw)
            pltpu.async_copy(
                row_v.at[pl.ds(slot * width, num_experts)],
                out_hbm.at[pl.ds((base + t) * num_experts, num_experts)],
                sem.at[slot])

        def tok_body(t, carry):
            slot = lax.rem(t, ring)
            pltpu.make_async_copy(
                row_v.at[pl.ds(slot * width, num_experts)],
                out_hbm.at[pl.ds(base * num_experts, num_experts)],
                sem.at[slot]).wait()
            for c in range(chunks):
                vi = idx_v[t - ring, pl.ds(c * lanes, lanes)] + slot * width
                plsc.store_scatter(row_v, [vi], zeros)
            fire(t, slot)
            return carry

        def prime_body(t, carry):
            fire(t, lax.rem(t, ring))
            return carry

        lax.fori_loop(0, ring, prime_body, 0)
        lax.fori_loop(ring, tpw, tok_body, 0)

        def drain_body(s, carry):
            pltpu.make_async_copy(
                row_v.at[pl.ds(s * width, num_experts)],
                out_hbm.at[pl.ds(base * num_experts, num_experts)],
                sem.at[s]).wait()
            return carry

        lax.fori_loop(0, ring, drain_body, 0)

    return scat


def _mlp_body(xk_ref, wd_ref, wu_ref, ws_ref, out_ref, acc_ref, *, ne_blocks):
    e = pl.program_id(0)
    t = pl.program_id(1)
    h = jax.lax.dot_general(
        xk_ref[...], wd_ref[...], (((1,), (1,)), ((), ())),
        preferred_element_type=jnp.float32)
    gelu = 0.5 * h * (1.0 + jax.lax.erf(h * (2.0 ** -0.5)))
    ws3 = ws_ref[...]
    gate = gelu * ws3.reshape(ws3.shape[0], ws3.shape[1] * ws3.shape[2])
    contrib = jax.lax.dot_general(
        gate.astype(jnp.bfloat16), wu_ref[...], (((1,), (0,)), ((), ())),
        preferred_element_type=jnp.float32)
    tb = xk_ref.shape[0]
    sl = pl.ds(t * tb, tb)

    @pl.when(e == 0)
    def _():
        acc_ref[sl, :] = contrib

    @pl.when(e > 0)
    def _():
        acc_ref[sl, :] = acc_ref[sl, :] + contrib

    @pl.when(e == ne_blocks - 1)
    def _():
        out_ref[...] = acc_ref[sl, :]


def kernel(x, shift_state, time_maa_k, time_maa_r, W_q, keys_p, w_down, w_up):
    del time_maa_r
    bsz, t_len, n_embd = x.shape
    heads, num_keys, _, dim_key = keys_p.shape
    num_experts = num_keys * num_keys

    new_shift_state = x[:, -1]
    xprev = jnp.concatenate([shift_state[:, None, :], x[:, :-1]], axis=1)
    x2 = x.reshape(bsz * t_len, n_embd)
    xp2 = xprev.reshape(bsz * t_len, n_embd)
    tmk = time_maa_k.reshape(1, n_embd)
    tokens = bsz * t_len

    wqr = W_q.reshape(2 * heads, dim_key, n_embd).astype(jnp.bfloat16)
    kt = keys_p.transpose(2, 0, 1, 3).reshape(
        2 * heads, num_keys, dim_key).astype(jnp.bfloat16)
    wd_b = w_down.astype(jnp.bfloat16)
    wu_b = w_up.astype(jnp.bfloat16)

    tb = 512 if tokens % 512 == 0 else tokens
    nt = tokens // tb
    xk = pl.pallas_call(
        _xk_body,
        grid=(nt,),
        in_specs=[
            pl.BlockSpec((tb, n_embd), lambda i: (i, 0)),
            pl.BlockSpec((tb, n_embd), lambda i: (i, 0)),
            pl.BlockSpec((1, n_embd), lambda i: (0, 0)),
        ],
        out_specs=pl.BlockSpec((tb, n_embd), lambda i: (i, 0)),
        out_shape=jax.ShapeDtypeStruct((tokens, n_embd), jnp.bfloat16),
    )(x2, xp2, tmk)

    sim = pl.pallas_call(
        _qsim_body,
        grid=(2 * heads, nt),
        in_specs=[
            pl.BlockSpec((tb, n_embd), lambda p, t: (t, 0)),
            pl.BlockSpec((1, dim_key, n_embd), lambda p, t: (p, 0, 0)),
            pl.BlockSpec((1, num_keys, dim_key), lambda p, t: (p, 0, 0)),
        ],
        out_specs=pl.BlockSpec((1, tb, num_keys), lambda p, t: (p, t, 0)),
        out_shape=jax.ShapeDtypeStruct((2 * heads, tokens, num_keys),
                                       jnp.float32),
        compiler_params=pltpu.CompilerParams(
            dimension_semantics=("arbitrary", "arbitrary")),
    )(xk, wqr, kt)

    idx, wts = pl.pallas_call(
        functools.partial(_select_body, heads=heads, num_keys=num_keys),
        grid=(nt,),
        in_specs=[pl.BlockSpec((2 * heads, tb, num_keys), lambda i: (0, i, 0))],
        out_specs=[
            pl.BlockSpec((heads, tb, TOPK), lambda i: (0, i, 0)),
            pl.BlockSpec((heads, tb, TOPK), lambda i: (0, i, 0)),
        ],
        out_shape=[
            jax.ShapeDtypeStruct((heads, tokens, TOPK), jnp.int32),
            jax.ShapeDtypeStruct((heads, tokens, TOPK), jnp.float32),
        ],
    )(sim)

    slots = heads * TOPK
    idx2 = idx.transpose(1, 0, 2).reshape(tokens, slots)
    wts2 = wts.transpose(1, 0, 2).reshape(tokens, slots)
    wts2 = pl.pallas_call(
        _dedup_body,
        grid=(nt,),
        in_specs=[
            pl.BlockSpec((tb, slots), lambda i: (i, 0)),
            pl.BlockSpec((tb, slots), lambda i: (i, 0)),
        ],
        out_specs=pl.BlockSpec((tb, slots), lambda i: (i, 0)),
        out_shape=jax.ShapeDtypeStruct((tokens, slots), jnp.float32),
    )(idx2, wts2)
    scat = _make_sc_scatter(tokens, slots, num_experts)
    wsum3 = scat(idx2, wts2).reshape(tokens, num_keys, num_keys)

    ne_blocks = 8
    eb = num_experts // ne_blocks
    out = pl.pallas_call(
        functools.partial(_mlp_body, ne_blocks=ne_blocks),
        grid=(ne_blocks, nt),
        in_specs=[
            pl.BlockSpec((tb, n_embd), lambda e, t: (t, 0)),
            pl.BlockSpec((eb, n_embd), lambda e, t: (e, 0)),
            pl.BlockSpec((eb, n_embd), lambda e, t: (e, 0)),
            pl.BlockSpec((tb, eb // num_keys, num_keys),
                         lambda e, t: (t, e, 0)),
        ],
        out_specs=pl.BlockSpec(
            (tb, n_embd),
            lambda e, t: (jnp.where(e == ne_blocks - 1, t, 0), 0)),
        out_shape=jax.ShapeDtypeStruct((tokens, n_embd), jnp.float32),
        scratch_shapes=[pltpu.VMEM((tokens, n_embd), jnp.float32)],
        compiler_params=pltpu.CompilerParams(
            dimension_semantics=("arbitrary", "arbitrary")),
    )(xk, wd_b, wu_b, wsum3)

    return out.reshape(bsz, t_len, n_embd), new_shift_state


# speedup vs baseline: 1.0745x; 1.0001x over previous
import functools

import jax
import jax.numpy as jnp
from jax import lax
from jax.experimental import pallas as pl
from jax.experimental.pallas import tpu as pltpu
from jax.experimental.pallas import tpu_sc as plsc

TOPK = 8


def _xk_body(x_ref, xp_ref, tmk_ref, xk_ref):
    x = x_ref[...]
    xk_ref[...] = (x + (xp_ref[...] - x) * tmk_ref[...]).astype(jnp.bfloat16)


def _qsim_body(xk_ref, wq_ref, kt_ref, sim_ref):
    q = jax.lax.dot_general(
        xk_ref[...], wq_ref[0], (((1,), (1,)), ((), ())),
        preferred_element_type=jnp.float32)
    sim_ref[...] = jax.lax.dot_general(
        q.astype(jnp.bfloat16), kt_ref[0], (((1,), (1,)), ((), ())),
        preferred_element_type=jnp.float32)[None]


def _select_body(sim_ref, idx_ref, w_ref, *, heads, num_keys):
    s = sim_ref[...]
    tb = s.shape[1]
    iota = jax.lax.broadcasted_iota(jnp.int32, s.shape, 2)
    vs, is_ = [], []
    v = s
    for _ in range(TOPK):
        m = jnp.max(v, axis=-1, keepdims=True)
        am = jnp.min(jnp.where(v >= m, iota, jnp.int32(1 << 30)),
                     axis=-1, keepdims=True)
        vs.append(m)
        is_.append(am)
        v = jnp.where(iota == am, -jnp.inf, v)
    sv = jnp.concatenate(vs, -1)
    si = jnp.concatenate(is_, -1)
    sx, ix = sv[:heads], si[:heads]
    sy, iy = sv[heads:], si[heads:]
    alls = (sx[..., :, None] + sy[..., None, :]).reshape(heads, tb, TOPK * TOPK)
    iota2 = jax.lax.broadcasted_iota(
        jnp.int32, (heads, tb, TOPK * TOPK), 2)
    iota8 = jax.lax.broadcasted_iota(jnp.int32, (heads, tb, TOPK), 2)
    vs, ps = [], []
    v = alls
    for _ in range(TOPK):
        m = jnp.max(v, axis=-1, keepdims=True)
        am = jnp.min(jnp.where(v >= m, iota2, jnp.int32(1 << 30)),
                     axis=-1, keepdims=True)
        hi = am >> 3
        lo = am & 7
        selx = jnp.sum(jnp.where(iota8 == hi, ix, 0), axis=-1, keepdims=True)
        sely = jnp.sum(jnp.where(iota8 == lo, iy, 0), axis=-1, keepdims=True)
        vs.append(m)
        ps.append(selx * num_keys + sely)
        v = jnp.where(iota2 == am, -jnp.inf, v)
    scores = jnp.concatenate(vs, -1)
    sel = jnp.concatenate(ps, -1)
    mx = jnp.max(scores, -1, keepdims=True)
    e = jnp.exp(scores - mx)
    w = e / jnp.sum(e, -1, keepdims=True)
    idx_ref[...] = sel
    w_ref[...] = w


def _dedup_body(idx_ref, w_ref, out_ref):
    idx = idx_ref[...]
    w = w_ref[...]
    eq = idx[:, :, None] == idx[:, None, :]
    out_ref[...] = jnp.sum(jnp.where(eq, w[:, None, :], 0.0), axis=2)


def _make_sc_scatter(tokens, slots, num_experts):
    info = plsc.get_sparse_core_info()
    nw = info.num_cores * info.num_subcores
    lanes = info.num_lanes
    tpw = tokens // nw
    width = num_experts + lanes
    chunks = slots // lanes
    ring = 8
    mesh = plsc.VectorSubcoreMesh(core_axis_name="c", subcore_axis_name="s")

    @functools.partial(
        pl.kernel, mesh=mesh,
        compiler_params=pltpu.CompilerParams(needs_layout_passes=False),
        out_type=jax.ShapeDtypeStruct((tokens * num_experts,), jnp.float32),
        scratch_types=[
            pltpu.VMEM((tpw, slots), jnp.int32),
            pltpu.VMEM((tpw, slots), jnp.float32),
            pltpu.VMEM((ring * width,), jnp.float32),
            pltpu.SemaphoreType.DMA((ring,)),
        ],
    )
    def scat(idx_hbm, w_hbm, out_hbm, idx_v, w_v, row_v, sem):
        wid = lax.axis_index("s") * info.num_cores + lax.axis_index("c")
        base = wid * tpw
        pltpu.sync_copy(idx_hbm.at[pl.ds(base, tpw)], idx_v)
        pltpu.sync_copy(w_hbm.at[pl.ds(base, tpw)], w_v)
        zeros = jnp.zeros((lanes,), jnp.float32)

        def zero_body(i, carry):
            row_v[pl.ds(i * lanes, lanes)] = zeros
            return carry

        lax.fori_loop(0, ring * width // lanes, zero_body, 0)

        def fire(t, slot):
            for c in range(chunks):
                vi = idx_v[t, pl.ds(c * lanes, lanes)] + slot * width
                vw = w_v[t, pl.ds(c * lanes, lanes)]
                plsc.store_scatter(row_v, [vi], vw)
            pltpu.async_copy(
                row_v.at[pl.ds(slot * width, num_experts)],
                out_hbm.at[pl.ds((base + t) * num_experts, num_experts)],
                sem.at[slot])

        def tok_body(t, carry):
            slot = lax.rem(t, ring)
            pltpu.make_async_copy(
                row_v.at[pl.ds(slot * width, num_experts)],
                out_hbm.at[pl.ds(base * num_experts, num_experts)],
                sem.at[slot]).wait()
            for c in range(chunks):
                vi = idx_v[t - ring, pl.ds(c * lanes, lanes)] + slot * width
                plsc.store_scatter(row_v, [vi], zeros)
            fire(t, slot)
            return carry

        def prime_body(t, carry):
            fire(t, lax.rem(t, ring))
            return carry

        lax.fori_loop(0, ring, prime_body, 0)
        lax.fori_loop(ring, tpw, tok_body, 0)

        def drain_body(s, carry):
            pltpu.make_async_copy(
                row_v.at[pl.ds(s * width, num_experts)],
                out_hbm.at[pl.ds(base * num_experts, num_experts)],
                sem.at[s]).wait()
            return carry

        lax.fori_loop(0, ring, drain_body, 0)

    return scat


def _mlp_body(xk_ref, wd_ref, wu_ref, ws_ref, out_ref, acc_ref, *, ne_blocks):
    e = pl.program_id(0)
    t = pl.program_id(1)
    h = jax.lax.dot_general(
        xk_ref[...], wd_ref[...], (((1,), (1,)), ((), ())),
        preferred_element_type=jnp.float32)
    gelu = 0.5 * h * (1.0 + jax.lax.erf(h * (2.0 ** -0.5)))
    ws3 = ws_ref[...]
    gate = gelu * ws3.reshape(ws3.shape[0], ws3.shape[1] * ws3.shape[2])
    contrib = jax.lax.dot_general(
        gate.astype(jnp.bfloat16), wu_ref[...], (((1,), (0,)), ((), ())),
        preferred_element_type=jnp.float32)
    tb = xk_ref.shape[0]
    sl = pl.ds(t * tb, tb)

    @pl.when(e == 0)
    def _():
        acc_ref[sl, :] = contrib

    @pl.when(e > 0)
    def _():
        acc_ref[sl, :] = acc_ref[sl, :] + contrib

    @pl.when(e == ne_blocks - 1)
    def _():
        out_ref[...] = acc_ref[sl, :]


def kernel(x, shift_state, time_maa_k, time_maa_r, W_q, keys_p, w_down, w_up):
    del time_maa_r
    bsz, t_len, n_embd = x.shape
    heads, num_keys, _, dim_key = keys_p.shape
    num_experts = num_keys * num_keys

    new_shift_state = x[:, -1]
    xprev = jnp.concatenate([shift_state[:, None, :], x[:, :-1]], axis=1)
    x2 = x.reshape(bsz * t_len, n_embd)
    xp2 = xprev.reshape(bsz * t_len, n_embd)
    tmk = time_maa_k.reshape(1, n_embd)
    tokens = bsz * t_len

    wqr = W_q.reshape(2 * heads, dim_key, n_embd).astype(jnp.bfloat16)
    kt = keys_p.transpose(2, 0, 1, 3).reshape(
        2 * heads, num_keys, dim_key).astype(jnp.bfloat16)
    wd_b = w_down.astype(jnp.bfloat16)
    wu_b = w_up.astype(jnp.bfloat16)

    tb = 512 if tokens % 512 == 0 else tokens
    nt = tokens // tb
    xk = pl.pallas_call(
        _xk_body,
        grid=(nt,),
        in_specs=[
            pl.BlockSpec((tb, n_embd), lambda i: (i, 0)),
            pl.BlockSpec((tb, n_embd), lambda i: (i, 0)),
            pl.BlockSpec((1, n_embd), lambda i: (0, 0)),
        ],
        out_specs=pl.BlockSpec((tb, n_embd), lambda i: (i, 0)),
        out_shape=jax.ShapeDtypeStruct((tokens, n_embd), jnp.bfloat16),
    )(x2, xp2, tmk)

    sim = pl.pallas_call(
        _qsim_body,
        grid=(2 * heads, nt),
        in_specs=[
            pl.BlockSpec((tb, n_embd), lambda p, t: (t, 0)),
            pl.BlockSpec((1, dim_key, n_embd), lambda p, t: (p, 0, 0)),
            pl.BlockSpec((1, num_keys, dim_key), lambda p, t: (p, 0, 0)),
        ],
        out_specs=pl.BlockSpec((1, tb, num_keys), lambda p, t: (p, t, 0)),
        out_shape=jax.ShapeDtypeStruct((2 * heads, tokens, num_keys),
                                       jnp.float32),
        compiler_params=pltpu.CompilerParams(
            dimension_semantics=("arbitrary", "arbitrary")),
    )(xk, wqr, kt)

    idx, wts = pl.pallas_call(
        functools.partial(_select_body, heads=heads, num_keys=num_keys),
        grid=(nt,),
        in_specs=[pl.BlockSpec((2 * heads, tb, num_keys), lambda i: (0, i, 0))],
        out_specs=[
            pl.BlockSpec((heads, tb, TOPK), lambda i: (0, i, 0)),
            pl.BlockSpec((heads, tb, TOPK), lambda i: (0, i, 0)),
        ],
        out_shape=[
            jax.ShapeDtypeStruct((heads, tokens, TOPK), jnp.int32),
            jax.ShapeDtypeStruct((heads, tokens, TOPK), jnp.float32),
        ],
    )(sim)

    slots = heads * TOPK
    idx2 = idx.transpose(1, 0, 2).reshape(tokens, slots)
    wts2 = wts.transpose(1, 0, 2).reshape(tokens, slots)
    wts2 = pl.pallas_call(
        _dedup_body,
        grid=(nt,),
        in_specs=[
            pl.BlockSpec((tb, slots), lambda i: (i, 0)),
            pl.BlockSpec((tb, slots), lambda i: (i, 0)),
        ],
        out_specs=pl.BlockSpec((tb, slots), lambda i: (i, 0)),
        out_shape=jax.ShapeDtypeStruct((tokens, slots), jnp.float32),
    )(idx2, wts2)
    scat = _make_sc_scatter(tokens, slots, num_experts)
    wsum3 = scat(idx2, wts2).reshape(tokens, num_keys, num_keys)

    ne_blocks = 8
    eb = num_experts // ne_blocks
    out = pl.pallas_call(
        functools.partial(_mlp_body, ne_blocks=ne_blocks),
        grid=(ne_blocks, nt),
        in_specs=[
            pl.BlockSpec((tb, n_embd), lambda e, t: (t, 0)),
            pl.BlockSpec((eb, n_embd), lambda e, t: (e, 0)),
            pl.BlockSpec((eb, n_embd), lambda e, t: (e, 0)),
            pl.BlockSpec((tb, eb // num_keys, num_keys),
                         lambda e, t: (t, e, 0)),
        ],
        out_specs=pl.BlockSpec(
            (tb, n_embd),
            lambda e, t: (jnp.where(e == ne_blocks - 1, t, 0), 0)),
        out_shape=jax.ShapeDtypeStruct((tokens, n_embd), jnp.float32),
        scratch_shapes=[pltpu.VMEM((tokens, n_embd), jnp.float32)],
        compiler_params=pltpu.CompilerParams(
            dimension_semantics=("arbitrary", "arbitrary")),
    )(xk, wd_b, wu_b, wsum3)

    return out.reshape(bsz, t_len, n_embd), new_shift_state
